# baseline (speedup 1.0000x reference)
import functools

import jax
import jax.numpy as jnp
from jax import lax
from jax.experimental import pallas as pl
from jax.experimental.pallas import tpu as pltpu
from jax.experimental.pallas import tpu_sc as plsc

N = 10000
E = 320000
D = 128
NS = 16
CH = 32
G = 8
NK = 640
EP = NS * NK * CH
NGRP = NK // G
NP = 10240
PADNODE = NP - 1
HR = NP // 2
AR = 5248
RPT = NP // NS
HRT = HR // NS
ART = AR // NS
HC = 5136


@functools.lru_cache(maxsize=None)
def _mesh():
    return plsc.VectorSubcoreMesh(
        core_axis_name="c", subcore_axis_name="s", num_cores=1)


def _clamp(v, r):
    if r == 0:
        return jnp.minimum(v, HR)
    return jnp.maximum(v, HR - 8) - (HR - 8)


@functools.lru_cache(maxsize=None)
def _deg_kernel():
    return pl.kernel(
        _deg_body,
        out_type=jax.ShapeDtypeStruct((NS * NP,), jnp.float32),
        mesh=_mesh(),
        compiler_params=pltpu.CompilerParams(needs_layout_passes=False),
        scratch_types=[
            pltpu.VMEM((G * CH // 16, 16), jnp.int32),
            pltpu.VMEM((HC,), jnp.float32),
        ],
    )


_GR = G * CH // 16


def _deg_body(dst_hbm, out_hbm, dstv, hist):
    s = lax.axis_index("s")
    zero = jnp.zeros((16,), jnp.float32)
    ones = jnp.ones((16,), jnp.float32)

    for r in range(2):
        def zrow(i, c):
            hist[pl.ds(i * 16, 16)] = zero
            return c
        lax.fori_loop(0, HC // 16, zrow, 0)

        def group(g, carry, r=r):
            pltpu.sync_copy(dst_hbm.at[s, pl.ds(g * _GR, _GR)], dstv)
            for j in range(_GR):
                m = _clamp(dstv[j, :], r)
                plsc.addupdate_scatter(hist, [m], ones)
            return carry

        lax.fori_loop(0, NGRP, group, 0)
        base = 0 if r == 0 else 8
        pltpu.sync_copy(hist.at[pl.ds(base, HR)],
                        out_hbm.at[pl.ds(s * NP + r * HR, HR)])


def _agg_body_factory(npasses):
    def body(*refs):
        tables = refs[:npasses]
        (src_hbm, dst_hbm, out_hbm, srcv, dstv, dstm,
         b0, b1, b2, b3, acc, gs0, gs1, gs2, gs3,
         ss0, ss1, ss2, ss3, isem) = refs[npasses:]
        s = lax.axis_index("s")
        bufs = (b0, b1, b2, b3)
        gsem = (gs0, gs1, gs2, gs3)
        ssem = (ss0, ss1, ss2, ss3)

        zero = jnp.zeros((16,), jnp.float32)

        def _zero_b0(i, c):
            def col(l, c2):
                b0[i, pl.ds(l * 16, 16)] = zero
                return c2
            return lax.fori_loop(0, D // 16, col, c)

        def load_idx(g):
            row = (g % 3) * G
            pltpu.async_copy(src_hbm.at[s, pl.ds(g * G, G)],
                             srcv.at[pl.ds(row, G)], isem)
            pltpu.async_copy(dst_hbm.at[s, pl.ds(g * G, G)],
                             dstv.at[pl.ds(row, G)], isem)

        def wait_idx(g):
            row = (g % 3) * G
            pltpu.make_async_copy(src_hbm.at[s, pl.ds(g * G, G)],
                                  srcv.at[pl.ds(row, G)], isem).wait()
            pltpu.make_async_copy(dst_hbm.at[s, pl.ds(g * G, G)],
                                  dstv.at[pl.ds(row, G)], isem).wait()

        def remap(g, r):
            base3 = (g % 3) * G
            base2 = (g % 2) * G
            for j in range(G):
                for l in range(CH // 16):
                    v = dstv[base3 + j, pl.ds(l * 16, 16)]
                    dstm[base2 + j, pl.ds(l * 16, 16)] = _clamp(v, r)

        def gather_start(table, g, j):
            b = j % 4
            pltpu.async_copy(table.at[srcv.at[(g % 3) * G + j]],
                             bufs[b], gsem[b])

        def gather_wait(table, g, j):
            b = j % 4
            pltpu.make_async_copy(table.at[srcv.at[(g % 3) * G + j]],
                                  bufs[b], gsem[b]).wait()

        def scat_start(g, j):
            b = j % 4
            pltpu.async_copy(bufs[b], acc.at[dstm.at[(g % 2) * G + j]],
                             ssem[b], add=True)

        def scat_wait(g, j):
            b = j % 4
            pltpu.make_async_copy(bufs[b], acc.at[dstm.at[(g % 2) * G + j]],
                                  ssem[b]).wait()

        def prev(g, j, lag):
            jj = j - lag
            if jj >= 0:
                return g, jj
            return g - 1, jj + G

        for p in range(npasses):
            table = tables[p]
            for r in range(2):
                lax.fori_loop(0, 8, _zero_b0, 0)
                for k in range(ART // 8):
                    pltpu.sync_copy(b0.at[pl.ds(0, 8)],
                                    acc.at[pl.ds(s * ART + k * 8, 8)])
                plsc.subcore_barrier()

                load_idx(0)
                wait_idx(0)
                load_idx(1)
                remap(0, r)
                for j in range(G):
                    if j >= 4:
                        scat_wait(0, j - 4)
                    gather_start(table, 0, j)
                    if j >= 2:
                        gather_wait(table, 0, j - 2)
                        scat_start(0, j - 2)

                def grp(g, carry, r=r, table=table):
                    wait_idx(g)

                    @pl.when(g + 1 < NGRP)
                    def _():
                        load_idx(g + 1)
                    remap(g, r)
                    for j in range(G):
                        g4, j4 = prev(g, j, 4)
                        scat_wait(g4, j4)
                        gather_start(table, g, j)
                        g2, j2 = prev(g, j, 2)
                        gather_wait(table, g2, j2)
                        scat_start(g2, j2)
                    return carry

                lax.fori_loop(1, NGRP, grp, 0)

                gl = NGRP - 1
                for j in (G - 2, G - 1):
                    gather_wait(table, gl, j)
                    scat_start(gl, j)
                for j in (G - 4, G - 3, G - 2, G - 1):
                    scat_wait(gl, j)

                plsc.subcore_barrier()
                base = 0 if r == 0 else 8
                pltpu.sync_copy(
                    acc.at[pl.ds(base + s * HRT, HRT)],
                    out_hbm.at[pl.ds(p * NP + r * HR + s * HRT, HRT)])
                plsc.subcore_barrier()
    return body


@functools.lru_cache(maxsize=None)
def _agg_kernel(npasses):
    return pl.kernel(
        _agg_body_factory(npasses),
        out_type=jax.ShapeDtypeStruct((npasses * NP, D), jnp.float32),
        mesh=_mesh(),
        scratch_types=[
            pltpu.VMEM((3 * G, CH), jnp.int32),
            pltpu.VMEM((3 * G, CH), jnp.int32),
            pltpu.VMEM((2 * G, CH), jnp.int32),
            pltpu.VMEM((CH, D), jnp.float32),
            pltpu.VMEM((CH, D), jnp.float32),
            pltpu.VMEM((CH, D), jnp.float32),
            pltpu.VMEM((CH, D), jnp.float32),
            pltpu.VMEM_SHARED((AR, D), jnp.float32),
            pltpu.SemaphoreType.DMA,
            pltpu.SemaphoreType.DMA,
            pltpu.SemaphoreType.DMA,
            pltpu.SemaphoreType.DMA,
            pltpu.SemaphoreType.DMA,
            pltpu.SemaphoreType.DMA,
            pltpu.SemaphoreType.DMA,
            pltpu.SemaphoreType.DMA,
            pltpu.SemaphoreType.DMA,
        ],
    )


def _dinv_from_counts(degc):
    deg = jnp.sum(degc.reshape(NS, NP), axis=0)
    return lax.rsqrt(deg[:N] + 1.0)


_PAD = NP - N


def _padded(h):
    return jnp.concatenate(
        [h, jnp.zeros((_PAD, h.shape[1]), jnp.float32)], axis=0)


def _tc1_body(x_ref, w1_ref, degc_ref, out_ref):
    dinv = _dinv_from_counts(degc_ref[...])
    h = jnp.dot(x_ref[...], w1_ref[...], preferred_element_type=jnp.float32)
    out_ref[...] = _padded(h * dinv[:, None])


def _tc2_body(tmp1_ref, g1_ref, degc_ref, b1_ref, w2_ref, t0_ref, t1_ref):
    dinv = _dinv_from_counts(degc_ref[...])
    out1 = jax.nn.relu(
        (tmp1_ref[:N, :] + g1_ref[:N, :]) * dinv[:, None] + b1_ref[...])
    h2 = jnp.dot(out1, w2_ref[...], preferred_element_type=jnp.float32)
    h2 = h2 * dinv[:, None]
    t0_ref[...] = _padded(h2[:, :D])
    t1_ref[...] = _padded(h2[:, D:])


def _tc3_body(tmp2_ref, t0_ref, t1_ref, degc_ref, b2_ref, out_ref):
    dinv = _dinv_from_counts(degc_ref[...])
    b2 = b2_ref[...]
    o0 = jax.nn.relu(
        (tmp2_ref[:N, :] + t0_ref[:N, :]) * dinv[:, None] + b2[:D])
    o1 = jax.nn.relu(
        (tmp2_ref[NP:NP + N, :] + t1_ref[:N, :]) * dinv[:, None] + b2[D:])
    out_ref[...] = jnp.concatenate(
        [jnp.mean(o0, axis=0), jnp.mean(o1, axis=0)])


_f32 = jnp.float32


def kernel(x, edge_index, W1, b1, W2, b2):
    src = jnp.concatenate(
        [edge_index[0], jnp.zeros((EP - E,), jnp.int32)])
    dst = jnp.concatenate(
        [edge_index[1], jnp.full((EP - E,), PADNODE, jnp.int32)])
    srce = src.reshape(NS, NK, CH)
    dste = dst.reshape(NS, NK, CH)
    dste16 = dst.reshape(NS, NK * CH // 16, 16)

    degc = _deg_kernel()(dste16)

    g1 = pl.pallas_call(
        _tc1_body,
        out_shape=jax.ShapeDtypeStruct((NP, D), _f32),
    )(x, W1, degc)

    tmp1 = _agg_kernel(1)(g1, srce, dste)

    t0, t1 = pl.pallas_call(
        _tc2_body,
        out_shape=(jax.ShapeDtypeStruct((NP, D), _f32),
                   jax.ShapeDtypeStruct((NP, D), _f32)),
    )(tmp1, g1, degc, b1, W2)

    tmp2 = _agg_kernel(2)(t0, t1, srce, dste)

    out = pl.pallas_call(
        _tc3_body,
        out_shape=jax.ShapeDtypeStruct((2 * D,), _f32),
    )(tmp2, t0, t1, degc, b2)
    return out


# speedup vs baseline: 1.0667x; 1.0667x over previous
import functools

import jax
import jax.numpy as jnp
from jax import lax
from jax.experimental import pallas as pl
from jax.experimental.pallas import tpu as pltpu
from jax.experimental.pallas import tpu_sc as plsc

N = 10000
E = 320000
D = 128
NS = 16
CH = 32
G = 8
NK = 640
EP = NS * NK * CH
NGRP = NK // G
NP = 10240
PADNODE = NP - 1
HR = NP // 2
AR = 5248
RPT = NP // NS
HRT = HR // NS
ART = AR // NS
HC = 5184


@functools.lru_cache(maxsize=None)
def _mesh():
    return plsc.VectorSubcoreMesh(
        core_axis_name="c", subcore_axis_name="s", num_cores=1)


DUMP = 64


def _clamp(v, r):
    spread = lax.bitwise_and(v, DUMP - 1)
    if r == 0:
        return jnp.where(v < HR, v, HR + spread)
    return jnp.where(v >= HR, v - HR + DUMP, spread)


@functools.lru_cache(maxsize=None)
def _deg_kernel():
    return pl.kernel(
        _deg_body,
        out_type=jax.ShapeDtypeStruct((NS * NP,), jnp.float32),
        mesh=_mesh(),
        compiler_params=pltpu.CompilerParams(needs_layout_passes=False),
        scratch_types=[
            pltpu.VMEM((G * CH // 16, 16), jnp.int32),
            pltpu.VMEM((HC,), jnp.float32),
        ],
    )


_GR = G * CH // 16


def _deg_body(dst_hbm, out_hbm, dstv, hist):
    s = lax.axis_index("s")
    zero = jnp.zeros((16,), jnp.float32)
    ones = jnp.ones((16,), jnp.float32)

    for r in range(2):
        def zrow(i, c):
            hist[pl.ds(i * 16, 16)] = zero
            return c
        lax.fori_loop(0, HC // 16, zrow, 0)

        def group(g, carry, r=r):
            pltpu.sync_copy(dst_hbm.at[s, pl.ds(g * _GR, _GR)], dstv)
            for j in range(_GR):
                m = _clamp(dstv[j, :], r)
                plsc.addupdate_scatter(hist, [m], ones)
            return carry

        lax.fori_loop(0, NGRP, group, 0)
        base = 0 if r == 0 else DUMP
        pltpu.sync_copy(hist.at[pl.ds(base, HR)],
                        out_hbm.at[pl.ds(s * NP + r * HR, HR)])


def _agg_body_factory(npasses):
    def body(*refs):
        tables = refs[:npasses]
        (src_hbm, dst_hbm, out_hbm, srcv, dstv, dstm,
         b0, b1, b2, b3, acc, gs0, gs1, gs2, gs3,
         ss0, ss1, ss2, ss3, isem) = refs[npasses:]
        s = lax.axis_index("s")
        bufs = (b0, b1, b2, b3)
        gsem = (gs0, gs1, gs2, gs3)
        ssem = (ss0, ss1, ss2, ss3)

        zero = jnp.zeros((16,), jnp.float32)

        def _zero_b0(i, c):
            def col(l, c2):
                b0[i, pl.ds(l * 16, 16)] = zero
                return c2
            return lax.fori_loop(0, D // 16, col, c)

        def load_idx(g):
            row = (g % 3) * G
            pltpu.async_copy(src_hbm.at[s, pl.ds(g * G, G)],
                             srcv.at[pl.ds(row, G)], isem)
            pltpu.async_copy(dst_hbm.at[s, pl.ds(g * G, G)],
                             dstv.at[pl.ds(row, G)], isem)

        def wait_idx(g):
            row = (g % 3) * G
            pltpu.make_async_copy(src_hbm.at[s, pl.ds(g * G, G)],
                                  srcv.at[pl.ds(row, G)], isem).wait()
            pltpu.make_async_copy(dst_hbm.at[s, pl.ds(g * G, G)],
                                  dstv.at[pl.ds(row, G)], isem).wait()

        def remap(g, r):
            base3 = (g % 3) * G
            base2 = (g % 2) * G
            for j in range(G):
                for l in range(CH // 16):
                    v = dstv[base3 + j, pl.ds(l * 16, 16)]
                    dstm[base2 + j, pl.ds(l * 16, 16)] = _clamp(v, r)

        def gather_start(table, g, j):
            b = j % 4
            pltpu.async_copy(table.at[srcv.at[(g % 3) * G + j]],
                             bufs[b], gsem[b])

        def gather_wait(table, g, j):
            b = j % 4
            pltpu.make_async_copy(table.at[srcv.at[(g % 3) * G + j]],
                                  bufs[b], gsem[b]).wait()

        def scat_start(g, j):
            b = j % 4
            pltpu.async_copy(bufs[b], acc.at[dstm.at[(g % 2) * G + j]],
                             ssem[b], add=True)

        def scat_wait(g, j):
            b = j % 4
            pltpu.make_async_copy(bufs[b], acc.at[dstm.at[(g % 2) * G + j]],
                                  ssem[b]).wait()

        def prev(g, j, lag):
            jj = j - lag
            if jj >= 0:
                return g, jj
            return g - 1, jj + G

        for p in range(npasses):
            table = tables[p]
            for r in range(2):
                lax.fori_loop(0, 8, _zero_b0, 0)
                for k in range(ART // 8):
                    pltpu.sync_copy(b0.at[pl.ds(0, 8)],
                                    acc.at[pl.ds(s * ART + k * 8, 8)])
                plsc.subcore_barrier()

                load_idx(0)
                wait_idx(0)
                load_idx(1)
                remap(0, r)
                for j in range(G):
                    if j >= 4:
                        scat_wait(0, j - 4)
                    gather_start(table, 0, j)
                    if j >= 2:
                        gather_wait(table, 0, j - 2)
                        scat_start(0, j - 2)

                def grp(g, carry, r=r, table=table):
                    wait_idx(g)

                    @pl.when(g + 1 < NGRP)
                    def _():
                        load_idx(g + 1)
                    remap(g, r)
                    for j in range(G):
                        g4, j4 = prev(g, j, 4)
                        scat_wait(g4, j4)
                        gather_start(table, g, j)
                        g2, j2 = prev(g, j, 2)
                        gather_wait(table, g2, j2)
                        scat_start(g2, j2)
                    return carry

                lax.fori_loop(1, NGRP, grp, 0)

                gl = NGRP - 1
                for j in (G - 2, G - 1):
                    gather_wait(table, gl, j)
                    scat_start(gl, j)
                for j in (G - 4, G - 3, G - 2, G - 1):
                    scat_wait(gl, j)

                plsc.subcore_barrier()
                base = 0 if r == 0 else DUMP
                pltpu.sync_copy(
                    acc.at[pl.ds(base + s * HRT, HRT)],
                    out_hbm.at[pl.ds(p * NP + r * HR + s * HRT, HRT)])
                plsc.subcore_barrier()
    return body


@functools.lru_cache(maxsize=None)
def _agg_kernel(npasses):
    return pl.kernel(
        _agg_body_factory(npasses),
        out_type=jax.ShapeDtypeStruct((npasses * NP, D), jnp.float32),
        mesh=_mesh(),
        scratch_types=[
            pltpu.VMEM((3 * G, CH), jnp.int32),
            pltpu.VMEM((3 * G, CH), jnp.int32),
            pltpu.VMEM((2 * G, CH), jnp.int32),
            pltpu.VMEM((CH, D), jnp.float32),
            pltpu.VMEM((CH, D), jnp.float32),
            pltpu.VMEM((CH, D), jnp.float32),
            pltpu.VMEM((CH, D), jnp.float32),
            pltpu.VMEM_SHARED((AR, D), jnp.float32),
            pltpu.SemaphoreType.DMA,
            pltpu.SemaphoreType.DMA,
            pltpu.SemaphoreType.DMA,
            pltpu.SemaphoreType.DMA,
            pltpu.SemaphoreType.DMA,
            pltpu.SemaphoreType.DMA,
            pltpu.SemaphoreType.DMA,
            pltpu.SemaphoreType.DMA,
            pltpu.SemaphoreType.DMA,
        ],
    )


def _dinv_from_counts(degc):
    deg = jnp.sum(degc.reshape(NS, NP), axis=0)
    return lax.rsqrt(deg[:N] + 1.0)


_PAD = NP - N


def _padded(h):
    return jnp.concatenate(
        [h, jnp.zeros((_PAD, h.shape[1]), jnp.float32)], axis=0)


def _tc1_body(x_ref, w1_ref, degc_ref, out_ref):
    dinv = _dinv_from_counts(degc_ref[...])
    h = jnp.dot(x_ref[...], w1_ref[...], preferred_element_type=jnp.float32)
    out_ref[...] = _padded(h * dinv[:, None])


def _tc2_body(tmp1_ref, g1_ref, degc_ref, b1_ref, w2_ref, t0_ref, t1_ref):
    dinv = _dinv_from_counts(degc_ref[...])
    out1 = jax.nn.relu(
        (tmp1_ref[:N, :] + g1_ref[:N, :]) * dinv[:, None] + b1_ref[...])
    h2 = jnp.dot(out1, w2_ref[...], preferred_element_type=jnp.float32)
    h2 = h2 * dinv[:, None]
    t0_ref[...] = _padded(h2[:, :D])
    t1_ref[...] = _padded(h2[:, D:])


def _tc3_body(tmp2_ref, t0_ref, t1_ref, degc_ref, b2_ref, out_ref):
    dinv = _dinv_from_counts(degc_ref[...])
    b2 = b2_ref[...]
    o0 = jax.nn.relu(
        (tmp2_ref[:N, :] + t0_ref[:N, :]) * dinv[:, None] + b2[:D])
    o1 = jax.nn.relu(
        (tmp2_ref[NP:NP + N, :] + t1_ref[:N, :]) * dinv[:, None] + b2[D:])
    out_ref[...] = jnp.concatenate(
        [jnp.mean(o0, axis=0), jnp.mean(o1, axis=0)])


_f32 = jnp.float32


def kernel(x, edge_index, W1, b1, W2, b2):
    src = jnp.concatenate(
        [edge_index[0], jnp.zeros((EP - E,), jnp.int32)])
    dst = jnp.concatenate(
        [edge_index[1], jnp.full((EP - E,), PADNODE, jnp.int32)])
    srce = src.reshape(NS, NK, CH)
    dste = dst.reshape(NS, NK, CH)
    dste16 = dst.reshape(NS, NK * CH // 16, 16)

    degc = _deg_kernel()(dste16)

    g1 = pl.pallas_call(
        _tc1_body,
        out_shape=jax.ShapeDtypeStruct((NP, D), _f32),
    )(x, W1, degc)

    tmp1 = _agg_kernel(1)(g1, srce, dste)

    t0, t1 = pl.pallas_call(
        _tc2_body,
        out_shape=(jax.ShapeDtypeStruct((NP, D), _f32),
                   jax.ShapeDtypeStruct((NP, D), _f32)),
    )(tmp1, g1, degc, b1, W2)

    tmp2 = _agg_kernel(2)(t0, t1, srce, dste)

    out = pl.pallas_call(
        _tc3_body,
        out_shape=jax.ShapeDtypeStruct((2 * D,), _f32),
    )(tmp2, t0, t1, degc, b2)
    return out


# speedup vs baseline: 1.1553x; 1.0830x over previous
import functools

import jax
import jax.numpy as jnp
from jax import lax
from jax.experimental import pallas as pl
from jax.experimental.pallas import tpu as pltpu
from jax.experimental.pallas import tpu_sc as plsc

N = 10000
E = 320000
D = 128
NS = 16
CH = 32
G = 8
NK = 640
EP = NS * NK * CH
NGRP = NK // G
NP = 10240
PADNODE = NP - 1
HR = NP // 2
AR = 5248
RPT = NP // NS
HRT = HR // NS
ART = AR // NS
HC = 5184
BLK = 512
CAPB = 41
CAPE = CAPB * BLK


@functools.lru_cache(maxsize=None)
def _mesh():
    return plsc.VectorSubcoreMesh(
        core_axis_name="c", subcore_axis_name="s", num_cores=1)


DUMP = 64


def _clamp(v, r):
    spread = lax.bitwise_and(v, DUMP - 1)
    if r == 0:
        return jnp.where(v < HR, v, HR + spread)
    return jnp.where(v >= HR, v - HR + DUMP, spread)


@functools.lru_cache(maxsize=None)
def _deg_kernel():
    return pl.kernel(
        _deg_body,
        out_type=jax.ShapeDtypeStruct((NS * NP,), jnp.float32),
        mesh=_mesh(),
        compiler_params=pltpu.CompilerParams(needs_layout_passes=False),
        scratch_types=[
            pltpu.VMEM((G * CH // 16, 16), jnp.int32),
            pltpu.VMEM((HC,), jnp.float32),
        ],
    )


_GR = G * CH // 16


def _deg_body(dst_hbm, out_hbm, dstv, hist):
    s = lax.axis_index("s")
    zero = jnp.zeros((16,), jnp.float32)
    ones = jnp.ones((16,), jnp.float32)

    for r in range(2):
        def zrow(i, c):
            hist[pl.ds(i * 16, 16)] = zero
            return c
        lax.fori_loop(0, HC // 16, zrow, 0)

        def group(g, carry, r=r):
            pltpu.sync_copy(dst_hbm.at[s, pl.ds(g * _GR, _GR)], dstv)
            for j in range(_GR):
                m = _clamp(dstv[j, :], r)
                plsc.addupdate_scatter(hist, [m], ones)
            return carry

        lax.fori_loop(0, NGRP, group, 0)
        base = 0 if r == 0 else DUMP
        pltpu.sync_copy(hist.at[pl.ds(base, HR)],
                        out_hbm.at[pl.ds(s * NP + r * HR, HR)])


@functools.lru_cache(maxsize=None)
def _bucket_kernel():
    return pl.kernel(
        _bucket_body,
        out_type=(jax.ShapeDtypeStruct((NS * 2 * CAPE,), jnp.int32),
                  jax.ShapeDtypeStruct((NS * 2 * CAPE,), jnp.int32),
                  jax.ShapeDtypeStruct((NS * 32,), jnp.int32)),
        mesh=_mesh(),
        compiler_params=pltpu.CompilerParams(needs_layout_passes=False),
        scratch_types=[
            pltpu.VMEM((G * CH // 16, 16), jnp.int32),
            pltpu.VMEM((G * CH // 16, 16), jnp.int32),
            pltpu.VMEM((BLK + 16,), jnp.int32),
            pltpu.VMEM((BLK + 16,), jnp.int32),
            pltpu.VMEM((BLK + 16,), jnp.int32),
            pltpu.VMEM((BLK + 16,), jnp.int32),
            pltpu.VMEM((16,), jnp.int32),
        ],
    )


def _bucket_body(src_hbm, dst_hbm, bsrc_hbm, bdst_hbm, cnt_hbm,
                 srcv, dstv, sb0, db0, sb1, db1, cv):
    s = lax.axis_index("s")
    base0 = s * 2 * CAPE
    base1 = base0 + CAPE

    def group(g, carry):
        c0, c1, n0, n1 = carry
        pltpu.sync_copy(src_hbm.at[s, pl.ds(g * _GR, _GR)], srcv)
        pltpu.sync_copy(dst_hbm.at[s, pl.ds(g * _GR, _GR)], dstv)
        for i in range(_GR):
            sv = srcv[i, :]
            dv = dstv[i, :]
            m0 = dv < HR
            m1 = jnp.logical_not(m0)
            d1 = dv - (HR - DUMP)
            plsc.store_compressed(sb0.at[pl.ds(c0, 16)], sv, mask=m0)
            plsc.store_compressed(db0.at[pl.ds(c0, 16)], dv, mask=m0)
            plsc.store_compressed(sb1.at[pl.ds(c1, 16)], sv, mask=m1)
            plsc.store_compressed(db1.at[pl.ds(c1, 16)], d1, mask=m1)
            k0 = jnp.sum(m0.astype(jnp.int32))
            c0 = c0 + k0
            c1 = c1 + (16 - k0)
            full0 = c0 >= BLK
            full1 = c1 >= BLK

            @pl.when(full0)
            def _(n0=n0):
                pltpu.sync_copy(sb0.at[pl.ds(0, BLK)],
                                bsrc_hbm.at[pl.ds(base0 + n0 * BLK, BLK)])
                pltpu.sync_copy(db0.at[pl.ds(0, BLK)],
                                bdst_hbm.at[pl.ds(base0 + n0 * BLK, BLK)])
                sb0[pl.ds(0, 16)] = sb0[pl.ds(BLK, 16)]
                db0[pl.ds(0, 16)] = db0[pl.ds(BLK, 16)]

            @pl.when(full1)
            def _(n1=n1):
                pltpu.sync_copy(sb1.at[pl.ds(0, BLK)],
                                bsrc_hbm.at[pl.ds(base1 + n1 * BLK, BLK)])
                pltpu.sync_copy(db1.at[pl.ds(0, BLK)],
                                bdst_hbm.at[pl.ds(base1 + n1 * BLK, BLK)])
                sb1[pl.ds(0, 16)] = sb1[pl.ds(BLK, 16)]
                db1[pl.ds(0, 16)] = db1[pl.ds(BLK, 16)]

            c0 = jnp.where(full0, c0 - BLK, c0)
            c1 = jnp.where(full1, c1 - BLK, c1)
            n0 = n0 + full0.astype(jnp.int32)
            n1 = n1 + full1.astype(jnp.int32)
        return (c0, c1, n0, n1)

    c0, c1, n0, n1 = lax.fori_loop(
        0, NGRP, group, (jnp.int32(0), jnp.int32(0),
                         jnp.int32(0), jnp.int32(0)))

    dsrc = jnp.zeros((16,), jnp.int32)
    dd0 = jnp.full((16,), HR, jnp.int32)
    dd1 = jnp.zeros((16,), jnp.int32)
    for k in range(BLK // 16):
        o0 = jnp.minimum(c0 + k * 16, BLK)
        o1 = jnp.minimum(c1 + k * 16, BLK)
        sb0[pl.ds(o0, 16)] = dsrc
        db0[pl.ds(o0, 16)] = dd0
        sb1[pl.ds(o1, 16)] = dsrc
        db1[pl.ds(o1, 16)] = dd1
    pltpu.sync_copy(sb0.at[pl.ds(0, BLK)],
                    bsrc_hbm.at[pl.ds(base0 + n0 * BLK, BLK)])
    pltpu.sync_copy(db0.at[pl.ds(0, BLK)],
                    bdst_hbm.at[pl.ds(base0 + n0 * BLK, BLK)])
    pltpu.sync_copy(sb1.at[pl.ds(0, BLK)],
                    bsrc_hbm.at[pl.ds(base1 + n1 * BLK, BLK)])
    pltpu.sync_copy(db1.at[pl.ds(0, BLK)],
                    bdst_hbm.at[pl.ds(base1 + n1 * BLK, BLK)])
    n0 = n0 + 1
    n1 = n1 + 1
    cv[pl.ds(0, 16)] = jnp.broadcast_to(n0, (16,)).astype(jnp.int32)
    pltpu.sync_copy(cv, cnt_hbm.at[pl.ds(s * 32, 16)])
    cv[pl.ds(0, 16)] = jnp.broadcast_to(n1, (16,)).astype(jnp.int32)
    pltpu.sync_copy(cv, cnt_hbm.at[pl.ds(s * 32 + 16, 16)])


def _agg_body_factory(npasses):
    def body(*refs):
        tables = refs[:npasses]
        (src_hbm, dst_hbm, cnt_hbm, out_hbm, srcv, dstv, cntv,
         b0, b1, b2, b3, acc, gs0, gs1, gs2, gs3,
         ss0, ss1, ss2, ss3, isem) = refs[npasses:]
        s = lax.axis_index("s")
        bufs = (b0, b1, b2, b3)
        gsem = (gs0, gs1, gs2, gs3)
        ssem = (ss0, ss1, ss2, ss3)

        zero = jnp.zeros((16,), jnp.float32)

        def _zero_b0(i, c):
            def col(l, c2):
                b0[i, pl.ds(l * 16, 16)] = zero
                return c2
            return lax.fori_loop(0, D // 16, col, c)

        def make_ops(r):
            def load_idx(g):
                row = (g % 3) * G
                pltpu.async_copy(src_hbm.at[s, r, pl.ds(g * G, G)],
                                 srcv.at[pl.ds(row, G)], isem)
                pltpu.async_copy(dst_hbm.at[s, r, pl.ds(g * G, G)],
                                 dstv.at[pl.ds(row, G)], isem)

            def wait_idx(g):
                row = (g % 3) * G
                pltpu.make_async_copy(src_hbm.at[s, r, pl.ds(g * G, G)],
                                      srcv.at[pl.ds(row, G)], isem).wait()
                pltpu.make_async_copy(dst_hbm.at[s, r, pl.ds(g * G, G)],
                                      dstv.at[pl.ds(row, G)], isem).wait()
            return load_idx, wait_idx

        def gather_start(table, g, j):
            b = j % 4
            pltpu.async_copy(table.at[srcv.at[(g % 3) * G + j]],
                             bufs[b], gsem[b])

        def gather_wait(table, g, j):
            b = j % 4
            pltpu.make_async_copy(table.at[srcv.at[(g % 3) * G + j]],
                                  bufs[b], gsem[b]).wait()

        def scat_start(g, j):
            b = j % 4
            pltpu.async_copy(bufs[b], acc.at[dstv.at[(g % 3) * G + j]],
                             ssem[b], add=True)

        def scat_wait(g, j):
            b = j % 4
            pltpu.make_async_copy(bufs[b], acc.at[dstv.at[(g % 3) * G + j]],
                                  ssem[b]).wait()

        def prev(g, j, lag):
            jj = j - lag
            if jj >= 0:
                return g, jj
            return g - 1, jj + G

        for p in range(npasses):
            table = tables[p]
            for r in range(2):
                load_idx, wait_idx = make_ops(r)
                pltpu.sync_copy(cnt_hbm.at[pl.ds(s * 32 + r * 16, 16)], cntv)
                ngroups = cntv[pl.ds(0, 16)][0] * 2
                lax.fori_loop(0, 8, _zero_b0, 0)
                for k in range(ART // 8):
                    pltpu.sync_copy(b0.at[pl.ds(0, 8)],
                                    acc.at[pl.ds(s * ART + k * 8, 8)])
                plsc.subcore_barrier()

                load_idx(0)
                wait_idx(0)
                load_idx(1)
                for j in range(G):
                    if j >= 4:
                        scat_wait(0, j - 4)
                    gather_start(table, 0, j)
                    if j >= 2:
                        gather_wait(table, 0, j - 2)
                        scat_start(0, j - 2)

                def grp(g, carry, table=table, ngroups=ngroups,
                        load_idx=load_idx, wait_idx=wait_idx):
                    wait_idx(g)

                    @pl.when(g + 1 < ngroups)
                    def _():
                        load_idx(g + 1)
                    for j in range(G):
                        g4, j4 = prev(g, j, 4)
                        scat_wait(g4, j4)
                        gather_start(table, g, j)
                        g2, j2 = prev(g, j, 2)
                        gather_wait(table, g2, j2)
                        scat_start(g2, j2)
                    return carry

                lax.fori_loop(1, ngroups, grp, 0)

                gl = ngroups - 1
                for j in (G - 2, G - 1):
                    gather_wait(table, gl, j)
                    scat_start(gl, j)
                for j in (G - 4, G - 3, G - 2, G - 1):
                    scat_wait(gl, j)

                plsc.subcore_barrier()
                base = 0 if r == 0 else DUMP
                pltpu.sync_copy(
                    acc.at[pl.ds(base + s * HRT, HRT)],
                    out_hbm.at[pl.ds(p * NP + r * HR + s * HRT, HRT)])
                plsc.subcore_barrier()
    return body


@functools.lru_cache(maxsize=None)
def _agg_kernel(npasses):
    return pl.kernel(
        _agg_body_factory(npasses),
        out_type=jax.ShapeDtypeStruct((npasses * NP, D), jnp.float32),
        mesh=_mesh(),
        scratch_types=[
            pltpu.VMEM((3 * G, CH), jnp.int32),
            pltpu.VMEM((3 * G, CH), jnp.int32),
            pltpu.VMEM((16,), jnp.int32),
            pltpu.VMEM((CH, D), jnp.float32),
            pltpu.VMEM((CH, D), jnp.float32),
            pltpu.VMEM((CH, D), jnp.float32),
            pltpu.VMEM((CH, D), jnp.float32),
            pltpu.VMEM_SHARED((AR, D), jnp.float32),
            pltpu.SemaphoreType.DMA,
            pltpu.SemaphoreType.DMA,
            pltpu.SemaphoreType.DMA,
            pltpu.SemaphoreType.DMA,
            pltpu.SemaphoreType.DMA,
            pltpu.SemaphoreType.DMA,
            pltpu.SemaphoreType.DMA,
            pltpu.SemaphoreType.DMA,
            pltpu.SemaphoreType.DMA,
        ],
    )


def _dinv_from_counts(degc):
    deg = jnp.sum(degc.reshape(NS, NP), axis=0)
    return lax.rsqrt(deg[:N] + 1.0)


_PAD = NP - N


def _padded(h):
    return jnp.concatenate(
        [h, jnp.zeros((_PAD, h.shape[1]), jnp.float32)], axis=0)


def _tc1_body(x_ref, w1_ref, degc_ref, out_ref):
    dinv = _dinv_from_counts(degc_ref[...])
    h = jnp.dot(x_ref[...], w1_ref[...], preferred_element_type=jnp.float32)
    out_ref[...] = _padded(h * dinv[:, None])


def _tc2_body(tmp1_ref, g1_ref, degc_ref, b1_ref, w2_ref, t0_ref, t1_ref):
    dinv = _dinv_from_counts(degc_ref[...])
    out1 = jax.nn.relu(
        (tmp1_ref[:N, :] + g1_ref[:N, :]) * dinv[:, None] + b1_ref[...])
    h2 = jnp.dot(out1, w2_ref[...], preferred_element_type=jnp.float32)
    h2 = h2 * dinv[:, None]
    t0_ref[...] = _padded(h2[:, :D])
    t1_ref[...] = _padded(h2[:, D:])


def _tc3_body(tmp2_ref, t0_ref, t1_ref, degc_ref, b2_ref, out_ref):
    dinv = _dinv_from_counts(degc_ref[...])
    b2 = b2_ref[...]
    o0 = jax.nn.relu(
        (tmp2_ref[:N, :] + t0_ref[:N, :]) * dinv[:, None] + b2[:D])
    o1 = jax.nn.relu(
        (tmp2_ref[NP:NP + N, :] + t1_ref[:N, :]) * dinv[:, None] + b2[D:])
    out_ref[...] = jnp.concatenate(
        [jnp.mean(o0, axis=0), jnp.mean(o1, axis=0)])


_f32 = jnp.float32


def kernel(x, edge_index, W1, b1, W2, b2):
    src = jnp.concatenate(
        [edge_index[0], jnp.zeros((EP - E,), jnp.int32)])
    dst = jnp.concatenate(
        [edge_index[1], jnp.full((EP - E,), PADNODE, jnp.int32)])
    srce16 = src.reshape(NS, NK * CH // 16, 16)
    dste16 = dst.reshape(NS, NK * CH // 16, 16)

    bsrc, bdst, cnts = _bucket_kernel()(srce16, dste16)
    bsrc4 = bsrc.reshape(NS, 2, CAPE // CH, CH)
    bdst4 = bdst.reshape(NS, 2, CAPE // CH, CH)

    degc = _deg_kernel()(dste16)

    g1 = pl.pallas_call(
        _tc1_body,
        out_shape=jax.ShapeDtypeStruct((NP, D), _f32),
    )(x, W1, degc)

    tmp1 = _agg_kernel(1)(g1, bsrc4, bdst4, cnts)

    t0, t1 = pl.pallas_call(
        _tc2_body,
        out_shape=(jax.ShapeDtypeStruct((NP, D), _f32),
                   jax.ShapeDtypeStruct((NP, D), _f32)),
    )(tmp1, g1, degc, b1, W2)

    tmp2 = _agg_kernel(2)(t0, t1, bsrc4, bdst4, cnts)

    out = pl.pallas_call(
        _tc3_body,
        out_shape=jax.ShapeDtypeStruct((2 * D,), _f32),
    )(tmp2, t0, t1, degc, b2)
    return out
